# Initial kernel scaffold; baseline (speedup 1.0000x reference)
import jax
import jax.numpy as jnp
from jax.experimental import pallas as pl


def kernel(z, codebook):
    raise NotImplementedError("write your pallas kernel here")


# speedup vs baseline: 1.5328x; 1.5328x over previous
import jax
import jax.numpy as jnp
from jax.experimental import pallas as pl
from jax.experimental.pallas import tpu as pltpu

_N_CLUSTERS = 1024
_EMBED_DIM = 32
_BETA = 0.25
_EPS = 1e-05

_BLOCK = 1024


def _vq_body(z_ref, cb_ref, zq_ref, idx_ref, commit_ref, cbloss_ref,
             cluster_ref, perp_ref, acc_ref, counts_ref):
    i = pl.program_id(0)
    nblk = pl.num_programs(0)
    z = z_ref[...]
    cb = cb_ref[...]

    z2 = jnp.sum(z * z, axis=-1, keepdims=True)
    e2 = jnp.sum(cb * cb, axis=-1)
    ze = jax.lax.dot_general(z, cb, (((1,), (1,)), ((), ())),
                             preferred_element_type=jnp.float32)
    dist = z2 + e2[None, :] - 2.0 * ze

    minval = jnp.min(dist, axis=-1, keepdims=True)
    iota = jax.lax.broadcasted_iota(jnp.int32, dist.shape, 1)
    idx = jnp.min(jnp.where(dist == minval, iota, _N_CLUSTERS), axis=-1)
    idx_ref[0, 0, :] = idx

    onehot = (iota == idx[:, None]).astype(jnp.float32)
    zq = jax.lax.dot_general(onehot, cb, (((1,), (0,)), ((), ())),
                             preferred_element_type=jnp.float32)
    zq_ref[...] = z + (zq - z)

    d = zq - z
    bsum = jnp.sum(d * d)
    csum = jnp.sum(onehot, axis=0)[None, :]

    @pl.when(i == 0)
    def _init():
        acc_ref[0, 0] = bsum
        counts_ref[...] = csum

    @pl.when(i > 0)
    def _accum():
        acc_ref[0, 0] += bsum
        counts_ref[...] += csum

    @pl.when(i == nblk - 1)
    def _finalize():
        total_sq = acc_ref[0, 0]
        loss = total_sq / jnp.float32(nblk * _BLOCK * _EMBED_DIM)
        commit_ref[0, 0] = loss
        cbloss_ref[0, 0] = loss
        cluster_ref[0, 0] = loss + _BETA * loss
        counts = counts_ref[...]
        probs = counts / (jnp.sum(counts) + _EPS)
        perp_ref[0, 0] = jnp.exp(-jnp.sum(probs * jnp.log(probs + _EPS)))


def _vq_pallas(z_flat, codebook, interpret=False):
    n = z_flat.shape[0]
    nblk = n // _BLOCK
    k = codebook.shape[0]
    out_shapes = (
        jax.ShapeDtypeStruct((n, _EMBED_DIM), jnp.float32),
        jax.ShapeDtypeStruct((nblk, 1, _BLOCK), jnp.int32),
        jax.ShapeDtypeStruct((1, 1), jnp.float32),
        jax.ShapeDtypeStruct((1, 1), jnp.float32),
        jax.ShapeDtypeStruct((1, 1), jnp.float32),
        jax.ShapeDtypeStruct((1, 1), jnp.float32),
    )
    grid = (nblk,)
    return pl.pallas_call(
        _vq_body,
        grid=grid,
        in_specs=[
            pl.BlockSpec((_BLOCK, _EMBED_DIM), lambda i: (i, 0)),
            pl.BlockSpec((k, _EMBED_DIM), lambda i: (0, 0)),
        ],
        out_specs=(
            pl.BlockSpec((_BLOCK, _EMBED_DIM), lambda i: (i, 0)),
            pl.BlockSpec((1, 1, _BLOCK), lambda i: (i, 0, 0)),
            pl.BlockSpec((1, 1), lambda i: (0, 0), memory_space=pltpu.SMEM),
            pl.BlockSpec((1, 1), lambda i: (0, 0), memory_space=pltpu.SMEM),
            pl.BlockSpec((1, 1), lambda i: (0, 0), memory_space=pltpu.SMEM),
            pl.BlockSpec((1, 1), lambda i: (0, 0), memory_space=pltpu.SMEM),
        ),
        out_shape=out_shapes,
        scratch_shapes=[
            pltpu.SMEM((1, 1), jnp.float32),
            pltpu.VMEM((1, k), jnp.float32),
        ],
        interpret=interpret,
    )(z_flat, codebook)


def kernel(z, codebook):
    B, V, P, D = z.shape
    z_flat = z.reshape(-1, D)
    zq_st, idx, commit, cbloss, cluster, perp = _vq_pallas(z_flat, codebook)
    return (
        zq_st.reshape(B, V, P, D),
        commit[0, 0],
        cbloss[0, 0],
        cluster[0, 0],
        perp[0, 0],
        idx.reshape(B, V, P),
    )
